# Initial kernel scaffold; baseline (speedup 1.0000x reference)
import jax
import jax.numpy as jnp
from jax.experimental import pallas as pl


def kernel(feat_P, feat_A, feat_state, edge_p2p, edge_p2a, edge_a2p, edge_a2a, edge_p2s, edge_a2s, W_P, b_P, W_A, b_A, W_p2p, b_p2p, W_p2a, b_p2a, W_a2p, b_a2p, W_a2a, b_a2a, W_p2s, b_p2s, W_a2s, b_a2s, W_in, b_in, attn_src_p2p, attn_dst_p2p, attn_src_p2a, attn_dst_p2a, attn_src_a2p, attn_dst_a2p, attn_src_a2a, attn_dst_a2a, attn_src_p2s, attn_dst_p2s, attn_src_a2s, attn_dst_a2s):
    raise NotImplementedError("write your pallas kernel here")


# speedup vs baseline: 32.0085x; 32.0085x over previous
import functools

import jax
import jax.numpy as jnp
import numpy as np
from jax import lax
from jax.experimental import pallas as pl
from jax.experimental.pallas import tpu as pltpu
from jax.experimental.pallas import tpu_sc as plsc

N = 10000
E = 320000
H = 4
D = 32
F = 128
SRC_COLS = 144
AD_COLS = 16
NC = 2
NS = 16
CHUNK = 80
EDGES_PER_TILE = E // (NC * NS)
NCHUNK = EDGES_PER_TILE // CHUNK
RU = 8
NU = N // RU
BM = 1000


def _matmul_kernel(x_ref, w_ref, b_ref, o_ref):
    o_ref[...] = (
        jnp.dot(x_ref[0], w_ref[0], preferred_element_type=jnp.float32)
        + b_ref[0]
    )[None]


def _tc_matmul(feats, W_eff, b_eff):
    grid = (3, N // BM)
    return pl.pallas_call(
        _matmul_kernel,
        grid=grid,
        in_specs=[
            pl.BlockSpec((1, BM, 128), lambda t, i: (t, i, 0)),
            pl.BlockSpec((1, 128, 512), lambda t, i: (t, 0, 0)),
            pl.BlockSpec((1, 1, 512), lambda t, i: (t, 0, 0)),
        ],
        out_specs=pl.BlockSpec((1, BM, 512), lambda t, i: (t, i, 0)),
        out_shape=jax.ShapeDtypeStruct((3, N, 512), jnp.float32),
    )(feats, W_eff, b_eff)


def _edge_body(edges, src_tab, ad_tab, num_out, s_out,
               idxs_v, idxd_v, rows_v, ad_v, out_v, w_v, z_v, zs_v,
               acc_num, acc_s, sem1, sem2):
    c = lax.axis_index("c")
    s = lax.axis_index("s")
    n_units = (NU - s + NS - 1) // NS

    def zrow(i, _):
        for j in range(8):
            z_v[i, pl.ds(j * 16, 16)] = jnp.zeros((16,), jnp.float32)
        zs_v[i, :] = jnp.zeros((16,), jnp.float32)
        return 0
    lax.fori_loop(0, RU, zrow, 0)

    for r in range(6):
        def zunit(j, _):
            sl = pl.ds((s + j * NS) * RU, RU)
            pltpu.sync_copy(z_v, acc_num.at[sl, :])
            pltpu.sync_copy(zs_v, acc_s.at[sl, :])
            return 0
        lax.fori_loop(0, n_units, zunit, 0)
        plsc.subcore_barrier()

        base0 = c * (E // NC) + s * EDGES_PER_TILE

        def chunk_body(k, _):
            base = base0 + k * CHUNK
            pltpu.sync_copy(edges.at[pl.ds(r * 2 * E + base, CHUNK)], idxs_v)
            pltpu.sync_copy(edges.at[pl.ds(r * 2 * E + E + base, CHUNK)],
                            idxd_v)
            cp1 = pltpu.async_copy(src_tab.at[r].at[idxs_v], rows_v, sem1)
            cp2 = pltpu.async_copy(ad_tab.at[r].at[idxd_v], ad_v, sem2)
            cp1.wait()
            cp2.wait()

            def edge_one(i, _):
                e = rows_v[i, pl.ds(F, 16)] + ad_v[i, :]
                e = jnp.where(e > 0, e, 0.2 * e)
                w = jnp.exp(e)
                w_v[i, :] = w
                for j in range(8):
                    wj = w[j // 2]
                    out_v[i, pl.ds(j * 16, 16)] = rows_v[i, pl.ds(j * 16, 16)] * wj
                return 0
            lax.fori_loop(0, CHUNK, edge_one, 0)

            pltpu.sync_copy(out_v, acc_num.at[idxd_v], add=True)
            pltpu.sync_copy(w_v, acc_s.at[idxd_v], add=True)
            return 0
        lax.fori_loop(0, NCHUNK, chunk_body, 0)
        plsc.subcore_barrier()

        def cunit(j, _):
            sl = pl.ds((s + j * NS) * RU, RU)
            pltpu.sync_copy(acc_num.at[sl, :], num_out.at[r, c, sl, :])
            pltpu.sync_copy(acc_s.at[sl, :], s_out.at[r, c, sl, :])
            return 0
        lax.fori_loop(0, n_units, cunit, 0)


_edge_kernel = functools.partial(
    pl.kernel,
    out_type=(
        jax.ShapeDtypeStruct((6, NC, N, F), jnp.float32),
        jax.ShapeDtypeStruct((6, NC, N, AD_COLS), jnp.float32),
    ),
    mesh=plsc.VectorSubcoreMesh(core_axis_name="c", subcore_axis_name="s",
                                num_cores=NC, num_subcores=NS),
    compiler_params=pltpu.CompilerParams(use_tc_tiling_on_sc=False),
    scratch_types=[
        pltpu.VMEM((CHUNK,), jnp.int32),
        pltpu.VMEM((CHUNK,), jnp.int32),
        pltpu.VMEM((CHUNK, SRC_COLS), jnp.float32),
        pltpu.VMEM((CHUNK, AD_COLS), jnp.float32),
        pltpu.VMEM((CHUNK, F), jnp.float32),
        pltpu.VMEM((CHUNK, AD_COLS), jnp.float32),
        pltpu.VMEM((RU, F), jnp.float32),
        pltpu.VMEM((RU, AD_COLS), jnp.float32),
        pltpu.VMEM_SHARED((N, F), jnp.float32),
        pltpu.VMEM_SHARED((N, AD_COLS), jnp.float32),
        pltpu.SemaphoreType.DMA,
        pltpu.SemaphoreType.DMA,
    ],
)(_edge_body)


def _combine_kernel(num_ref, s_ref, whin_ref, b16_ref, op_ref, oa_ref, os_ref):
    b16 = b16_ref[...]
    ft = []
    for r in range(6):
        n = num_ref[r, 0] + num_ref[r, 1]
        sv = s_ref[r, 0] + s_ref[r, 1]
        recip = 1.0 / (sv + 1e-9)
        ft.append(n * jnp.dot(recip, b16, preferred_element_type=jnp.float32))
    op_ref[...] = jax.nn.relu(ft[0] + ft[2])
    oa_ref[...] = jax.nn.relu(ft[1] + ft[3])
    os_ref[...] = jax.nn.relu(ft[4] + ft[5] + whin_ref[...])


def _tc_combine(num, svals, whin, b16):
    grid = (N // BM,)
    out_sds = jax.ShapeDtypeStruct((N, F), jnp.float32)
    return pl.pallas_call(
        _combine_kernel,
        grid=grid,
        in_specs=[
            pl.BlockSpec((6, NC, BM, F), lambda i: (0, 0, i, 0)),
            pl.BlockSpec((6, NC, BM, AD_COLS), lambda i: (0, 0, i, 0)),
            pl.BlockSpec((BM, F), lambda i: (i, 0)),
            pl.BlockSpec((AD_COLS, F), lambda i: (0, 0)),
        ],
        out_specs=[
            pl.BlockSpec((BM, F), lambda i: (i, 0)),
            pl.BlockSpec((BM, F), lambda i: (i, 0)),
            pl.BlockSpec((BM, F), lambda i: (i, 0)),
        ],
        out_shape=[out_sds, out_sds, out_sds],
    )(num, svals, whin, b16)


def _attn_cols(W, b, attn):
    a = attn[0]
    d = a.shape[1]
    v = jnp.einsum('khd,hd->kh', W.reshape(128, H, d), a)
    vb = jnp.einsum('hd,hd->h', b.reshape(H, d), a)
    return v, vb


def _type_block(W_rel, b_rel, attn_src):
    v, vb = _attn_cols(W_rel, b_rel, attn_src)
    z = jnp.zeros((128, 12), jnp.float32)
    zb = jnp.zeros((12,), jnp.float32)
    return (jnp.concatenate([W_rel, v, z], axis=1),
            jnp.concatenate([b_rel, vb, zb]))


def _ad_block(W_t, b_t, attn_dst):
    v, vb = _attn_cols(W_t, b_t, attn_dst)
    z = jnp.zeros((128, 12), jnp.float32)
    zb = jnp.zeros((12,), jnp.float32)
    return jnp.concatenate([v, z], axis=1), jnp.concatenate([vb, zb])


def kernel(feat_P, feat_A, feat_state, edge_p2p, edge_p2a, edge_a2p, edge_a2a, edge_p2s, edge_a2s, W_P, b_P, W_A, b_A, W_p2p, b_p2p, W_p2a, b_p2a, W_a2p, b_a2p, W_a2a, b_a2a, W_p2s, b_p2s, W_a2s, b_a2s, W_in, b_in, attn_src_p2p, attn_dst_p2p, attn_src_p2a, attn_dst_p2a, attn_src_a2p, attn_dst_a2p, attn_src_a2a, attn_dst_a2a, attn_src_p2s, attn_dst_p2s, attn_src_a2s, attn_dst_a2s):
    f32 = jnp.float32

    zpad = jnp.zeros((128, 48), f32)
    zpadb = jnp.zeros((48,), f32)
    zblk = jnp.zeros((128, 144), f32)
    zblkb = jnp.zeros((144,), f32)

    bp0, bbp0 = _type_block(W_p2p, b_p2p, attn_src_p2p)
    bp1, bbp1 = _type_block(W_p2a, b_p2a, attn_src_p2a)
    bp2, bbp2 = _type_block(W_p2s, b_p2s, attn_src_p2s)
    adP0, adbP0 = _ad_block(W_P, b_P, attn_dst_p2p)
    adP1, adbP1 = _ad_block(W_P, b_P, attn_dst_a2p)
    WeP = jnp.concatenate([bp0, bp1, bp2, adP0, adP1, zpad], axis=1)
    beP = jnp.concatenate([bbp0, bbp1, bbp2, adbP0, adbP1, zpadb])

    ba0, bba0 = _type_block(W_a2p, b_a2p, attn_src_a2p)
    ba1, bba1 = _type_block(W_a2a, b_a2a, attn_src_a2a)
    ba2, bba2 = _type_block(W_a2s, b_a2s, attn_src_a2s)
    adA0, adbA0 = _ad_block(W_A, b_A, attn_dst_p2a)
    adA1, adbA1 = _ad_block(W_A, b_A, attn_dst_a2a)
    WeA = jnp.concatenate([ba0, ba1, ba2, adA0, adA1, zpad], axis=1)
    beA = jnp.concatenate([bba0, bba1, bba2, adbA0, adbA1, zpadb])

    bs0 = jnp.concatenate([W_in, jnp.zeros((128, 16), f32)], axis=1)
    bbs0 = jnp.concatenate([b_in, jnp.zeros((16,), f32)])
    adS0, adbS0 = _ad_block(W_in, b_in, attn_dst_p2s)
    adS1, adbS1 = _ad_block(W_in, b_in, attn_dst_a2s)
    WeS = jnp.concatenate([bs0, zblk, zblk, adS0, adS1, zpad], axis=1)
    beS = jnp.concatenate([bbs0, zblkb, zblkb, adbS0, adbS1, zpadb])

    W_eff = jnp.stack([WeP, WeA, WeS])
    b_eff = jnp.stack([beP, beA, beS])[:, None, :]
    feats = jnp.stack([feat_P, feat_A, feat_state])

    big = _tc_matmul(feats, W_eff, b_eff)
    bigP, bigA, bigS = big[0], big[1], big[2]

    src_tab = jnp.stack([
        bigP[:, 0:144], bigP[:, 144:288],
        bigA[:, 0:144], bigA[:, 144:288],
        bigP[:, 288:432], bigA[:, 288:432],
    ])
    ad_tab = jnp.stack([
        bigP[:, 432:448], bigA[:, 432:448],
        bigP[:, 448:464], bigA[:, 448:464],
        bigS[:, 432:448], bigS[:, 448:464],
    ])
    whin = bigS[:, 0:128]

    edges = jnp.stack([edge_p2p, edge_p2a, edge_a2p,
                       edge_a2a, edge_p2s, edge_a2s]).reshape(-1)

    num, svals = _edge_kernel(edges, src_tab, ad_tab)

    b16 = np.zeros((AD_COLS, F), np.float32)
    for h in range(H):
        b16[h, h * D:(h + 1) * D] = 1.0
    out_P, out_A, out_S = _tc_combine(num, svals, whin, jnp.asarray(b16))

    return (out_P.reshape(N, H, D),
            out_A.reshape(N, H, D),
            out_S.reshape(N, H, D))


# speedup vs baseline: 47.7216x; 1.4909x over previous
import functools

import jax
import jax.numpy as jnp
import numpy as np
from jax import lax
from jax.experimental import pallas as pl
from jax.experimental.pallas import tpu as pltpu
from jax.experimental.pallas import tpu_sc as plsc

N = 10000
E = 320000
H = 4
D = 32
F = 128
SRC_COLS = 144
AD_COLS = 16
NC = 2
NS = 16
CHUNK = 64
CPT = 158
NPREF = 2
NPAD = N + 16
IDXW = (CPT + NPREF) * 2 * CHUNK
RU = 8
NU = N // RU
BM = 1000


def _matmul_kernel(x_ref, w_ref, b_ref, o_ref):
    o_ref[...] = (
        jnp.dot(x_ref[0], w_ref[0], preferred_element_type=jnp.float32)
        + b_ref[0]
    )[None]


def _tc_matmul(feats, W_eff, b_eff):
    grid = (3, N // BM)
    return pl.pallas_call(
        _matmul_kernel,
        grid=grid,
        in_specs=[
            pl.BlockSpec((1, BM, 128), lambda t, i: (t, i, 0)),
            pl.BlockSpec((1, 128, 512), lambda t, i: (t, 0, 0)),
            pl.BlockSpec((1, 1, 512), lambda t, i: (t, 0, 0)),
        ],
        out_specs=pl.BlockSpec((1, BM, 512), lambda t, i: (t, i, 0)),
        out_shape=jax.ShapeDtypeStruct((3, N, 512), jnp.float32),
    )(feats, W_eff, b_eff)


def _edge_body(edges, src_tab, ad_tab, nums_out,
               ib0, ib1, idxd0, idxd1, rows0, rows1, ad0, ad1, out0, out1,
               z_v, acc, isem0, isem1, semg0, semg1):
    c = lax.axis_index("c")
    s = lax.axis_index("s")
    n_units = (NU - s + NS - 1) // NS
    bufs = ((ib0, rows0, ad0, out0, idxd0, isem0, semg0),
            (ib1, rows1, ad1, out1, idxd1, isem1, semg1))

    def zrow(i, _):
        for j in range(SRC_COLS // 16):
            z_v[i, pl.ds(j * 16, 16)] = jnp.zeros((16,), jnp.float32)
        return 0
    lax.fori_loop(0, RU, zrow, 0)

    def idx_dma(r, kk, ib_b, isem_b):
        base = ((r * NC + c) * NS + s) * IDXW + kk * 2 * CHUNK
        pltpu.async_copy(edges.at[pl.ds(base, 2 * CHUNK)], ib_b, isem_b)

    def idx_wait(ib_b, isem_b):
        pltpu.make_async_copy(edges.at[pl.ds(0, 2 * CHUNK)],
                              ib_b, isem_b).wait()

    def gather(r, ib_b, rows_b, ad_b, semg_b):
        pltpu.async_copy(src_tab.at[r].at[ib_b.at[pl.ds(0, CHUNK)]],
                         rows_b, semg_b)
        pltpu.async_copy(ad_tab.at[r].at[ib_b.at[pl.ds(CHUNK, CHUNK)]],
                         ad_b, semg_b)

    def gather_wait(r, ib_b, rows_b, ad_b, semg_b):
        pltpu.make_async_copy(src_tab.at[r].at[ib_b.at[pl.ds(0, CHUNK)]],
                              rows_b, semg_b).wait()
        pltpu.make_async_copy(ad_tab.at[r].at[ib_b.at[pl.ds(CHUNK, CHUNK)]],
                              ad_b, semg_b).wait()

    for r in range(6):
        def zunit(j, _):
            sl = pl.ds((s + j * NS) * RU, RU)
            pltpu.sync_copy(z_v, acc.at[sl, :])
            return 0
        lax.fori_loop(0, n_units, zunit, 0)
        plsc.subcore_barrier()

        idx_dma(r, 0, ib0, isem0)
        idx_dma(r, 1, ib1, isem1)
        idx_wait(ib0, isem0)
        gather(r, ib0, rows0, ad0, semg0)

        def pair_body(j, _):
            for b in range(2):
                ib_b, rows_b, ad_b, out_b, idxd_b, isem_b, semg_b = bufs[b]
                ib_n, rows_n, ad_n, out_n, idxd_n, isem_n, semg_n = bufs[1 - b]
                k = 2 * j + b
                idx_wait(ib_n, isem_n)
                gather(r, ib_n, rows_n, ad_n, semg_n)
                gather_wait(r, ib_b, rows_b, ad_b, semg_b)

                for jj in range(CHUNK // 16):
                    idxd_b[pl.ds(jj * 16, 16)] = (
                        ib_b[pl.ds(CHUNK + jj * 16, 16)])

                def edge_one(i, _):
                    e = rows_b[i, pl.ds(F, 16)] + ad_b[i, :]
                    e = jnp.where(e > 0, e, 0.2 * e)
                    w = jnp.exp(e)
                    out_b[i, pl.ds(F, 16)] = w
                    for jj in range(8):
                        out_b[i, pl.ds(jj * 16, 16)] = (
                            rows_b[i, pl.ds(jj * 16, 16)] * w[jj // 2])
                    return 0
                lax.fori_loop(0, CHUNK, edge_one, 0)

                pltpu.sync_copy(out_b, acc.at[idxd_b], add=True)
                idx_dma(r, k + 2, ib_b, isem_b)
            return 0
        lax.fori_loop(0, CPT // 2, pair_body, 0)

        idx_wait(ib1, isem1)
        gather_wait(r, ib0, rows0, ad0, semg0)
        plsc.subcore_barrier()

        def cunit(j, _):
            sl = pl.ds((s + j * NS) * RU, RU)
            pltpu.sync_copy(acc.at[sl, :], nums_out.at[r, c, sl, :])
            return 0
        lax.fori_loop(0, n_units, cunit, 0)


_edge_kernel = functools.partial(
    pl.kernel,
    out_type=jax.ShapeDtypeStruct((6, NC, N, SRC_COLS), jnp.float32),
    mesh=plsc.VectorSubcoreMesh(core_axis_name="c", subcore_axis_name="s",
                                num_cores=NC, num_subcores=NS),
    compiler_params=pltpu.CompilerParams(use_tc_tiling_on_sc=False),
    scratch_types=[
        pltpu.VMEM((2 * CHUNK,), jnp.int32),
        pltpu.VMEM((2 * CHUNK,), jnp.int32),
        pltpu.VMEM((CHUNK,), jnp.int32),
        pltpu.VMEM((CHUNK,), jnp.int32),
        pltpu.VMEM((CHUNK, SRC_COLS), jnp.float32),
        pltpu.VMEM((CHUNK, SRC_COLS), jnp.float32),
        pltpu.VMEM((CHUNK, AD_COLS), jnp.float32),
        pltpu.VMEM((CHUNK, AD_COLS), jnp.float32),
        pltpu.VMEM((CHUNK, SRC_COLS), jnp.float32),
        pltpu.VMEM((CHUNK, SRC_COLS), jnp.float32),
        pltpu.VMEM((RU, SRC_COLS), jnp.float32),
        pltpu.VMEM_SHARED((NPAD, SRC_COLS), jnp.float32),
        pltpu.SemaphoreType.DMA,
        pltpu.SemaphoreType.DMA,
        pltpu.SemaphoreType.DMA,
        pltpu.SemaphoreType.DMA,
    ],
)(_edge_body)


def _combine_kernel(num_ref, whin_ref, b16_ref, op_ref, oa_ref, os_ref):
    b16 = b16_ref[...]
    ft = []
    for r in range(6):
        x = num_ref[r, 0] + num_ref[r, 1]
        n = x[:, 0:F]
        sv = x[:, F:SRC_COLS]
        recip = 1.0 / (sv + 1e-9)
        ft.append(n * jnp.dot(recip, b16, preferred_element_type=jnp.float32))
    op_ref[...] = jax.nn.relu(ft[0] + ft[2])
    oa_ref[...] = jax.nn.relu(ft[1] + ft[3])
    os_ref[...] = jax.nn.relu(ft[4] + ft[5] + whin_ref[...])


def _tc_combine(nums, whin, b16):
    grid = (N // BM,)
    out_sds = jax.ShapeDtypeStruct((N, F), jnp.float32)
    return pl.pallas_call(
        _combine_kernel,
        grid=grid,
        in_specs=[
            pl.BlockSpec((6, NC, BM, SRC_COLS), lambda i: (0, 0, i, 0)),
            pl.BlockSpec((BM, F), lambda i: (i, 0)),
            pl.BlockSpec((AD_COLS, F), lambda i: (0, 0)),
        ],
        out_specs=[
            pl.BlockSpec((BM, F), lambda i: (i, 0)),
            pl.BlockSpec((BM, F), lambda i: (i, 0)),
            pl.BlockSpec((BM, F), lambda i: (i, 0)),
        ],
        out_shape=[out_sds, out_sds, out_sds],
    )(nums, whin, b16)


def _attn_cols(W, b, attn):
    a = attn[0]
    d = a.shape[1]
    v = jnp.einsum('khd,hd->kh', W.reshape(128, H, d), a)
    vb = jnp.einsum('hd,hd->h', b.reshape(H, d), a)
    return v, vb


def _type_block(W_rel, b_rel, attn_src):
    v, vb = _attn_cols(W_rel, b_rel, attn_src)
    z = jnp.zeros((128, 12), jnp.float32)
    zb = jnp.zeros((12,), jnp.float32)
    return (jnp.concatenate([W_rel, v, z], axis=1),
            jnp.concatenate([b_rel, vb, zb]))


def _ad_block(W_t, b_t, attn_dst):
    v, vb = _attn_cols(W_t, b_t, attn_dst)
    z = jnp.zeros((128, 12), jnp.float32)
    zb = jnp.zeros((12,), jnp.float32)
    return jnp.concatenate([v, z], axis=1), jnp.concatenate([vb, zb])


def kernel(feat_P, feat_A, feat_state, edge_p2p, edge_p2a, edge_a2p, edge_a2a, edge_p2s, edge_a2s, W_P, b_P, W_A, b_A, W_p2p, b_p2p, W_p2a, b_p2a, W_a2p, b_a2p, W_a2a, b_a2a, W_p2s, b_p2s, W_a2s, b_a2s, W_in, b_in, attn_src_p2p, attn_dst_p2p, attn_src_p2a, attn_dst_p2a, attn_src_a2p, attn_dst_a2p, attn_src_a2a, attn_dst_a2a, attn_src_p2s, attn_dst_p2s, attn_src_a2s, attn_dst_a2s):
    f32 = jnp.float32

    zpad = jnp.zeros((128, 48), f32)
    zpadb = jnp.zeros((48,), f32)
    zblk = jnp.zeros((128, 144), f32)
    zblkb = jnp.zeros((144,), f32)

    bp0, bbp0 = _type_block(W_p2p, b_p2p, attn_src_p2p)
    bp1, bbp1 = _type_block(W_p2a, b_p2a, attn_src_p2a)
    bp2, bbp2 = _type_block(W_p2s, b_p2s, attn_src_p2s)
    adP0, adbP0 = _ad_block(W_P, b_P, attn_dst_p2p)
    adP1, adbP1 = _ad_block(W_P, b_P, attn_dst_a2p)
    WeP = jnp.concatenate([bp0, bp1, bp2, adP0, adP1, zpad], axis=1)
    beP = jnp.concatenate([bbp0, bbp1, bbp2, adbP0, adbP1, zpadb])

    ba0, bba0 = _type_block(W_a2p, b_a2p, attn_src_a2p)
    ba1, bba1 = _type_block(W_a2a, b_a2a, attn_src_a2a)
    ba2, bba2 = _type_block(W_a2s, b_a2s, attn_src_a2s)
    adA0, adbA0 = _ad_block(W_A, b_A, attn_dst_p2a)
    adA1, adbA1 = _ad_block(W_A, b_A, attn_dst_a2a)
    WeA = jnp.concatenate([ba0, ba1, ba2, adA0, adA1, zpad], axis=1)
    beA = jnp.concatenate([bba0, bba1, bba2, adbA0, adbA1, zpadb])

    bs0 = jnp.concatenate([W_in, jnp.zeros((128, 16), f32)], axis=1)
    bbs0 = jnp.concatenate([b_in, jnp.zeros((16,), f32)])
    adS0, adbS0 = _ad_block(W_in, b_in, attn_dst_p2s)
    adS1, adbS1 = _ad_block(W_in, b_in, attn_dst_a2s)
    WeS = jnp.concatenate([bs0, zblk, zblk, adS0, adS1, zpad], axis=1)
    beS = jnp.concatenate([bbs0, zblkb, zblkb, adbS0, adbS1, zpadb])

    W_eff = jnp.stack([WeP, WeA, WeS])
    b_eff = jnp.stack([beP, beA, beS])[:, None, :]
    feats = jnp.stack([feat_P, feat_A, feat_state])

    big = _tc_matmul(feats, W_eff, b_eff)
    bigP, bigA, bigS = big[0], big[1], big[2]

    src_tab = jnp.stack([
        bigP[:, 0:144], bigP[:, 144:288],
        bigA[:, 0:144], bigA[:, 144:288],
        bigP[:, 288:432], bigA[:, 288:432],
    ])
    ad_tab = jnp.stack([
        bigP[:, 432:448], bigA[:, 432:448],
        bigP[:, 448:464], bigA[:, 448:464],
        bigS[:, 432:448], bigS[:, 448:464],
    ])
    ad_tab = jnp.concatenate(
        [ad_tab, jnp.zeros((6, NPAD - N, AD_COLS), f32)], axis=1)
    whin = bigS[:, 0:128]

    edges = jnp.stack([edge_p2p, edge_p2a, edge_a2p,
                       edge_a2a, edge_p2s, edge_a2s])
    per_sc = E // NC
    npad_e = NS * CPT * CHUNK - per_sc
    dummy_vals = jnp.array([0, N], jnp.int32)
    eh = edges.reshape(6, 2, NC, per_sc)
    pad1 = jnp.broadcast_to(dummy_vals[None, :, None, None],
                            (6, 2, NC, npad_e))
    eh = jnp.concatenate([eh, pad1], axis=3)
    eh = eh.reshape(6, 2, NC, NS, CPT, CHUNK)
    pad2 = jnp.broadcast_to(dummy_vals[None, :, None, None, None, None],
                            (6, 2, NC, NS, NPREF, CHUNK))
    eh = jnp.concatenate([eh, pad2], axis=4)
    edges = eh.transpose(0, 2, 3, 4, 1, 5).reshape(-1)

    nums = _edge_kernel(edges, src_tab, ad_tab)

    b16 = np.zeros((AD_COLS, F), np.float32)
    for h in range(H):
        b16[h, h * D:(h + 1) * D] = 1.0
    out_P, out_A, out_S = _tc_combine(nums, whin, jnp.asarray(b16))

    return (out_P.reshape(N, H, D),
            out_A.reshape(N, H, D),
            out_S.reshape(N, H, D))


# speedup vs baseline: 51.5766x; 1.0808x over previous
import functools

import jax
import jax.numpy as jnp
import numpy as np
from jax import lax
from jax.experimental import pallas as pl
from jax.experimental.pallas import tpu as pltpu
from jax.experimental.pallas import tpu_sc as plsc

N = 10000
E = 320000
H = 4
D = 32
F = 128
SRC_COLS = 144
AD_COLS = 16
NC = 2
NS = 16
CHUNK = 64
CPT = 158
NPREF = 2
NPAD = N + 16
IDXW = (CPT + NPREF) * 2 * CHUNK
RU = 8
NU = N // RU
BM = 1000


def _matmul_kernel(x_ref, w_ref, b_ref, o_ref):
    o_ref[...] = (
        jnp.dot(x_ref[0], w_ref[0], preferred_element_type=jnp.float32)
        + b_ref[0]
    )[None]


def _tc_matmul(feats, W_eff, b_eff):
    grid = (3, N // BM)
    return pl.pallas_call(
        _matmul_kernel,
        grid=grid,
        in_specs=[
            pl.BlockSpec((1, BM, 128), lambda t, i: (t, i, 0)),
            pl.BlockSpec((1, 128, 512), lambda t, i: (t, 0, 0)),
            pl.BlockSpec((1, 1, 512), lambda t, i: (t, 0, 0)),
        ],
        out_specs=pl.BlockSpec((1, BM, 512), lambda t, i: (t, i, 0)),
        out_shape=jax.ShapeDtypeStruct((3, N, 512), jnp.float32),
    )(feats, W_eff, b_eff)


def _edge_body(edges, src_tab, ad_tab, nums_out,
               ib0, ib1, idxd0, idxd1, rows0, rows1, ad0, ad1, out0, out1,
               z_v, acc, isem0, isem1, semg0, semg1, sems0, sems1):
    c = lax.axis_index("c")
    s = lax.axis_index("s")
    n_units = (NU - s + NS - 1) // NS
    bufs = ((ib0, rows0, ad0, out0, idxd0, isem0, semg0, sems0),
            (ib1, rows1, ad1, out1, idxd1, isem1, semg1, sems1))

    def zrow(i, _):
        for j in range(SRC_COLS // 16):
            z_v[i, pl.ds(j * 16, 16)] = jnp.zeros((16,), jnp.float32)
        return 0
    lax.fori_loop(0, RU, zrow, 0)

    def idx_dma(r, kk, ib_b, isem_b):
        base = ((r * NC + c) * NS + s) * IDXW + kk * 2 * CHUNK
        pltpu.async_copy(edges.at[pl.ds(base, 2 * CHUNK)], ib_b, isem_b)

    def idx_wait(ib_b, isem_b):
        pltpu.make_async_copy(edges.at[pl.ds(0, 2 * CHUNK)],
                              ib_b, isem_b).wait()

    def gather(r, ib_b, rows_b, ad_b, semg_b):
        pltpu.async_copy(src_tab.at[r].at[ib_b.at[pl.ds(0, CHUNK)]],
                         rows_b, semg_b)
        pltpu.async_copy(ad_tab.at[r].at[ib_b.at[pl.ds(CHUNK, CHUNK)]],
                         ad_b, semg_b)

    def gather_wait(r, ib_b, rows_b, ad_b, semg_b):
        pltpu.make_async_copy(src_tab.at[r].at[ib_b.at[pl.ds(0, CHUNK)]],
                              rows_b, semg_b).wait()
        pltpu.make_async_copy(ad_tab.at[r].at[ib_b.at[pl.ds(CHUNK, CHUNK)]],
                              ad_b, semg_b).wait()

    def scatter_wait(out_b, idxd_b, sems_b):
        pltpu.make_async_copy(out_b, acc.at[idxd_b], sems_b).wait()

    for r in range(6):
        def zunit(j, _):
            sl = pl.ds((s + j * NS) * RU, RU)
            pltpu.sync_copy(z_v, acc.at[sl, :])
            return 0
        lax.fori_loop(0, n_units, zunit, 0)
        plsc.subcore_barrier()

        idx_dma(r, 0, ib0, isem0)
        idx_dma(r, 1, ib1, isem1)
        idx_wait(ib0, isem0)
        gather(r, ib0, rows0, ad0, semg0)

        def pair_body(j, _):
            for b in range(2):
                (ib_b, rows_b, ad_b, out_b, idxd_b, isem_b, semg_b,
                 sems_b) = bufs[b]
                (ib_n, rows_n, ad_n, out_n, idxd_n, isem_n, semg_n,
                 sems_n) = bufs[1 - b]
                k = 2 * j + b
                idx_wait(ib_n, isem_n)
                gather(r, ib_n, rows_n, ad_n, semg_n)
                gather_wait(r, ib_b, rows_b, ad_b, semg_b)

                @pl.when(j > 0)
                def _():
                    scatter_wait(out_b, idxd_b, sems_b)

                for jj in range(CHUNK // 16):
                    idxd_b[pl.ds(jj * 16, 16)] = (
                        ib_b[pl.ds(CHUNK + jj * 16, 16)])

                def edge_one(i, _):
                    e = rows_b[i, pl.ds(F, 16)] + ad_b[i, :]
                    e = jnp.where(e > 0, e, 0.2 * e)
                    w = jnp.exp(e)
                    out_b[i, pl.ds(F, 16)] = w
                    for jj in range(8):
                        out_b[i, pl.ds(jj * 16, 16)] = (
                            rows_b[i, pl.ds(jj * 16, 16)] * w[jj // 2])
                    return 0
                lax.fori_loop(0, CHUNK, edge_one, 0)

                pltpu.async_copy(out_b, acc.at[idxd_b], sems_b, add=True)
                idx_dma(r, k + 2, ib_b, isem_b)
            return 0
        lax.fori_loop(0, CPT // 2, pair_body, 0)

        idx_wait(ib1, isem1)
        gather_wait(r, ib0, rows0, ad0, semg0)
        scatter_wait(out0, idxd0, sems0)
        scatter_wait(out1, idxd1, sems1)
        plsc.subcore_barrier()

        def cunit(j, _):
            sl = pl.ds((s + j * NS) * RU, RU)
            pltpu.sync_copy(acc.at[sl, :], nums_out.at[r, c, sl, :])
            return 0
        lax.fori_loop(0, n_units, cunit, 0)


_edge_kernel = functools.partial(
    pl.kernel,
    out_type=jax.ShapeDtypeStruct((6, NC, N, SRC_COLS), jnp.float32),
    mesh=plsc.VectorSubcoreMesh(core_axis_name="c", subcore_axis_name="s",
                                num_cores=NC, num_subcores=NS),
    compiler_params=pltpu.CompilerParams(use_tc_tiling_on_sc=False),
    scratch_types=[
        pltpu.VMEM((2 * CHUNK,), jnp.int32),
        pltpu.VMEM((2 * CHUNK,), jnp.int32),
        pltpu.VMEM((CHUNK,), jnp.int32),
        pltpu.VMEM((CHUNK,), jnp.int32),
        pltpu.VMEM((CHUNK, SRC_COLS), jnp.float32),
        pltpu.VMEM((CHUNK, SRC_COLS), jnp.float32),
        pltpu.VMEM((CHUNK, AD_COLS), jnp.float32),
        pltpu.VMEM((CHUNK, AD_COLS), jnp.float32),
        pltpu.VMEM((CHUNK, SRC_COLS), jnp.float32),
        pltpu.VMEM((CHUNK, SRC_COLS), jnp.float32),
        pltpu.VMEM((RU, SRC_COLS), jnp.float32),
        pltpu.VMEM_SHARED((NPAD, SRC_COLS), jnp.float32),
        pltpu.SemaphoreType.DMA,
        pltpu.SemaphoreType.DMA,
        pltpu.SemaphoreType.DMA,
        pltpu.SemaphoreType.DMA,
        pltpu.SemaphoreType.DMA,
        pltpu.SemaphoreType.DMA,
    ],
)(_edge_body)


def _combine_kernel(num_ref, whin_ref, b16_ref, op_ref, oa_ref, os_ref):
    b16 = b16_ref[...]
    ft = []
    for r in range(6):
        x = num_ref[r, 0] + num_ref[r, 1]
        n = x[:, 0:F]
        sv = x[:, F:SRC_COLS]
        recip = 1.0 / (sv + 1e-9)
        ft.append(n * jnp.dot(recip, b16, preferred_element_type=jnp.float32))
    op_ref[...] = jax.nn.relu(ft[0] + ft[2])
    oa_ref[...] = jax.nn.relu(ft[1] + ft[3])
    os_ref[...] = jax.nn.relu(ft[4] + ft[5] + whin_ref[...])


def _tc_combine(nums, whin, b16):
    grid = (N // BM,)
    out_sds = jax.ShapeDtypeStruct((N, F), jnp.float32)
    return pl.pallas_call(
        _combine_kernel,
        grid=grid,
        in_specs=[
            pl.BlockSpec((6, NC, BM, SRC_COLS), lambda i: (0, 0, i, 0)),
            pl.BlockSpec((BM, F), lambda i: (i, 0)),
            pl.BlockSpec((AD_COLS, F), lambda i: (0, 0)),
        ],
        out_specs=[
            pl.BlockSpec((BM, F), lambda i: (i, 0)),
            pl.BlockSpec((BM, F), lambda i: (i, 0)),
            pl.BlockSpec((BM, F), lambda i: (i, 0)),
        ],
        out_shape=[out_sds, out_sds, out_sds],
    )(nums, whin, b16)


def _attn_cols(W, b, attn):
    a = attn[0]
    d = a.shape[1]
    v = jnp.einsum('khd,hd->kh', W.reshape(128, H, d), a)
    vb = jnp.einsum('hd,hd->h', b.reshape(H, d), a)
    return v, vb


def _type_block(W_rel, b_rel, attn_src):
    v, vb = _attn_cols(W_rel, b_rel, attn_src)
    z = jnp.zeros((128, 12), jnp.float32)
    zb = jnp.zeros((12,), jnp.float32)
    return (jnp.concatenate([W_rel, v, z], axis=1),
            jnp.concatenate([b_rel, vb, zb]))


def _ad_block(W_t, b_t, attn_dst):
    v, vb = _attn_cols(W_t, b_t, attn_dst)
    z = jnp.zeros((128, 12), jnp.float32)
    zb = jnp.zeros((12,), jnp.float32)
    return jnp.concatenate([v, z], axis=1), jnp.concatenate([vb, zb])


def kernel(feat_P, feat_A, feat_state, edge_p2p, edge_p2a, edge_a2p, edge_a2a, edge_p2s, edge_a2s, W_P, b_P, W_A, b_A, W_p2p, b_p2p, W_p2a, b_p2a, W_a2p, b_a2p, W_a2a, b_a2a, W_p2s, b_p2s, W_a2s, b_a2s, W_in, b_in, attn_src_p2p, attn_dst_p2p, attn_src_p2a, attn_dst_p2a, attn_src_a2p, attn_dst_a2p, attn_src_a2a, attn_dst_a2a, attn_src_p2s, attn_dst_p2s, attn_src_a2s, attn_dst_a2s):
    f32 = jnp.float32

    zpad = jnp.zeros((128, 48), f32)
    zpadb = jnp.zeros((48,), f32)
    zblk = jnp.zeros((128, 144), f32)
    zblkb = jnp.zeros((144,), f32)

    bp0, bbp0 = _type_block(W_p2p, b_p2p, attn_src_p2p)
    bp1, bbp1 = _type_block(W_p2a, b_p2a, attn_src_p2a)
    bp2, bbp2 = _type_block(W_p2s, b_p2s, attn_src_p2s)
    adP0, adbP0 = _ad_block(W_P, b_P, attn_dst_p2p)
    adP1, adbP1 = _ad_block(W_P, b_P, attn_dst_a2p)
    WeP = jnp.concatenate([bp0, bp1, bp2, adP0, adP1, zpad], axis=1)
    beP = jnp.concatenate([bbp0, bbp1, bbp2, adbP0, adbP1, zpadb])

    ba0, bba0 = _type_block(W_a2p, b_a2p, attn_src_a2p)
    ba1, bba1 = _type_block(W_a2a, b_a2a, attn_src_a2a)
    ba2, bba2 = _type_block(W_a2s, b_a2s, attn_src_a2s)
    adA0, adbA0 = _ad_block(W_A, b_A, attn_dst_p2a)
    adA1, adbA1 = _ad_block(W_A, b_A, attn_dst_a2a)
    WeA = jnp.concatenate([ba0, ba1, ba2, adA0, adA1, zpad], axis=1)
    beA = jnp.concatenate([bba0, bba1, bba2, adbA0, adbA1, zpadb])

    bs0 = jnp.concatenate([W_in, jnp.zeros((128, 16), f32)], axis=1)
    bbs0 = jnp.concatenate([b_in, jnp.zeros((16,), f32)])
    adS0, adbS0 = _ad_block(W_in, b_in, attn_dst_p2s)
    adS1, adbS1 = _ad_block(W_in, b_in, attn_dst_a2s)
    WeS = jnp.concatenate([bs0, zblk, zblk, adS0, adS1, zpad], axis=1)
    beS = jnp.concatenate([bbs0, zblkb, zblkb, adbS0, adbS1, zpadb])

    W_eff = jnp.stack([WeP, WeA, WeS])
    b_eff = jnp.stack([beP, beA, beS])[:, None, :]
    feats = jnp.stack([feat_P, feat_A, feat_state])

    big = _tc_matmul(feats, W_eff, b_eff)
    bigP, bigA, bigS = big[0], big[1], big[2]

    src_tab = jnp.stack([
        bigP[:, 0:144], bigP[:, 144:288],
        bigA[:, 0:144], bigA[:, 144:288],
        bigP[:, 288:432], bigA[:, 288:432],
    ])
    ad_tab = jnp.stack([
        bigP[:, 432:448], bigA[:, 432:448],
        bigP[:, 448:464], bigA[:, 448:464],
        bigS[:, 432:448], bigS[:, 448:464],
    ])
    ad_tab = jnp.concatenate(
        [ad_tab, jnp.zeros((6, NPAD - N, AD_COLS), f32)], axis=1)
    whin = bigS[:, 0:128]

    edges = jnp.stack([edge_p2p, edge_p2a, edge_a2p,
                       edge_a2a, edge_p2s, edge_a2s])
    per_sc = E // NC
    npad_e = NS * CPT * CHUNK - per_sc
    dummy_vals = jnp.array([0, N], jnp.int32)
    eh = edges.reshape(6, 2, NC, per_sc)
    pad1 = jnp.broadcast_to(dummy_vals[None, :, None, None],
                            (6, 2, NC, npad_e))
    eh = jnp.concatenate([eh, pad1], axis=3)
    eh = eh.reshape(6, 2, NC, NS, CPT, CHUNK)
    pad2 = jnp.broadcast_to(dummy_vals[None, :, None, None, None, None],
                            (6, 2, NC, NS, NPREF, CHUNK))
    eh = jnp.concatenate([eh, pad2], axis=4)
    edges = eh.transpose(0, 2, 3, 4, 1, 5).reshape(-1)

    nums = _edge_kernel(edges, src_tab, ad_tab)

    b16 = np.zeros((AD_COLS, F), np.float32)
    for h in range(H):
        b16[h, h * D:(h + 1) * D] = 1.0
    out_P, out_A, out_S = _tc_combine(nums, whin, jnp.asarray(b16))

    return (out_P.reshape(N, H, D),
            out_A.reshape(N, H, D),
            out_S.reshape(N, H, D))


# speedup vs baseline: 53.0331x; 1.0282x over previous
import functools

import jax
import jax.numpy as jnp
import numpy as np
from jax import lax
from jax.experimental import pallas as pl
from jax.experimental.pallas import tpu as pltpu
from jax.experimental.pallas import tpu_sc as plsc

N = 10000
E = 320000
H = 4
D = 32
F = 128
SRC_COLS = 144
AD_COLS = 16
NC = 2
NS = 16
CHUNK = 80
CPT = 126
NPREF = 2
NPAD = N + 16
IDXW = (CPT + NPREF) * 2 * CHUNK
RU = 8
NU = N // RU
BM = 1000


def _matmul_kernel(x_ref, w_ref, b_ref, o_ref):
    o_ref[...] = (
        jnp.dot(x_ref[0], w_ref[0], preferred_element_type=jnp.float32)
        + b_ref[0]
    )[None]


def _tc_matmul(feats, W_eff, b_eff):
    grid = (3, N // BM)
    return pl.pallas_call(
        _matmul_kernel,
        grid=grid,
        in_specs=[
            pl.BlockSpec((1, BM, 128), lambda t, i: (t, i, 0)),
            pl.BlockSpec((1, 128, 512), lambda t, i: (t, 0, 0)),
            pl.BlockSpec((1, 1, 512), lambda t, i: (t, 0, 0)),
        ],
        out_specs=pl.BlockSpec((1, BM, 512), lambda t, i: (t, i, 0)),
        out_shape=jax.ShapeDtypeStruct((3, N, 512), jnp.float32),
    )(feats, W_eff, b_eff)


def _edge_body(edges, src_tab, ad_tab, nums_out,
               ib0, ib1, idxd0, idxd1, rows0, rows1, ad0, ad1, out0,
               z_v, acc, isem0, isem1, semg0, semg1, sems0):
    c = lax.axis_index("c")
    s = lax.axis_index("s")
    n_units = (NU - s + NS - 1) // NS
    bufs = ((ib0, rows0, ad0, idxd0, isem0, semg0),
            (ib1, rows1, ad1, idxd1, isem1, semg1))

    def zrow(i, _):
        for j in range(SRC_COLS // 16):
            z_v[i, pl.ds(j * 16, 16)] = jnp.zeros((16,), jnp.float32)
        return 0
    lax.fori_loop(0, RU, zrow, 0)

    def idx_dma(r, kk, ib_b, isem_b):
        base = ((r * NC + c) * NS + s) * IDXW + kk * 2 * CHUNK
        pltpu.async_copy(edges.at[pl.ds(base, 2 * CHUNK)], ib_b, isem_b)

    def idx_wait(ib_b, isem_b):
        pltpu.make_async_copy(edges.at[pl.ds(0, 2 * CHUNK)],
                              ib_b, isem_b).wait()

    def gather(r, ib_b, rows_b, ad_b, semg_b):
        pltpu.async_copy(src_tab.at[r].at[ib_b.at[pl.ds(0, CHUNK)]],
                         rows_b, semg_b)
        pltpu.async_copy(ad_tab.at[r].at[ib_b.at[pl.ds(CHUNK, CHUNK)]],
                         ad_b, semg_b)

    def gather_wait(r, ib_b, rows_b, ad_b, semg_b):
        pltpu.make_async_copy(src_tab.at[r].at[ib_b.at[pl.ds(0, CHUNK)]],
                              rows_b, semg_b).wait()
        pltpu.make_async_copy(ad_tab.at[r].at[ib_b.at[pl.ds(CHUNK, CHUNK)]],
                              ad_b, semg_b).wait()

    def scatter_wait(out_b, idxd_b, sems_b):
        pltpu.make_async_copy(out_b, acc.at[idxd_b], sems_b).wait()

    for r in range(6):
        def zunit(j, _):
            sl = pl.ds((s + j * NS) * RU, RU)
            pltpu.sync_copy(z_v, acc.at[sl, :])
            return 0
        lax.fori_loop(0, n_units, zunit, 0)
        plsc.subcore_barrier()

        idx_dma(r, 0, ib0, isem0)
        idx_dma(r, 1, ib1, isem1)
        idx_wait(ib0, isem0)
        gather(r, ib0, rows0, ad0, semg0)

        def pair_body(j, _):
            for b in range(2):
                ib_b, rows_b, ad_b, idxd_b, isem_b, semg_b = bufs[b]
                ib_n, rows_n, ad_n, idxd_n, isem_n, semg_n = bufs[1 - b]
                k = 2 * j + b
                idx_wait(ib_n, isem_n)
                gather(r, ib_n, rows_n, ad_n, semg_n)
                gather_wait(r, ib_b, rows_b, ad_b, semg_b)

                for jj in range(CHUNK // 16):
                    idxd_b[pl.ds(jj * 16, 16)] = (
                        ib_b[pl.ds(CHUNK + jj * 16, 16)])

                if b == 1:
                    scatter_wait(out0, idxd_n, sems0)
                else:
                    @pl.when(j > 0)
                    def _():
                        scatter_wait(out0, idxd_n, sems0)

                def edge_one(i, _):
                    e = rows_b[i, pl.ds(F, 16)] + ad_b[i, :]
                    e = jnp.where(e > 0, e, 0.2 * e)
                    w = jnp.exp(e)
                    out0[i, pl.ds(F, 16)] = w
                    for jj in range(8):
                        out0[i, pl.ds(jj * 16, 16)] = (
                            rows_b[i, pl.ds(jj * 16, 16)] * w[jj // 2])
                    return 0
                lax.fori_loop(0, CHUNK, edge_one, 0)

                pltpu.async_copy(out0, acc.at[idxd_b], sems0, add=True)
                idx_dma(r, k + 2, ib_b, isem_b)
            return 0
        lax.fori_loop(0, CPT // 2, pair_body, 0)

        idx_wait(ib1, isem1)
        gather_wait(r, ib0, rows0, ad0, semg0)
        scatter_wait(out0, idxd1, sems0)
        plsc.subcore_barrier()

        def cunit(j, _):
            sl = pl.ds((s + j * NS) * RU, RU)
            pltpu.sync_copy(acc.at[sl, :], nums_out.at[r, c, sl, :])
            return 0
        lax.fori_loop(0, n_units, cunit, 0)


_edge_kernel = functools.partial(
    pl.kernel,
    out_type=jax.ShapeDtypeStruct((6, NC, N, SRC_COLS), jnp.float32),
    mesh=plsc.VectorSubcoreMesh(core_axis_name="c", subcore_axis_name="s",
                                num_cores=NC, num_subcores=NS),
    compiler_params=pltpu.CompilerParams(use_tc_tiling_on_sc=False),
    scratch_types=[
        pltpu.VMEM((2 * CHUNK,), jnp.int32),
        pltpu.VMEM((2 * CHUNK,), jnp.int32),
        pltpu.VMEM((CHUNK,), jnp.int32),
        pltpu.VMEM((CHUNK,), jnp.int32),
        pltpu.VMEM((CHUNK, SRC_COLS), jnp.float32),
        pltpu.VMEM((CHUNK, SRC_COLS), jnp.float32),
        pltpu.VMEM((CHUNK, AD_COLS), jnp.float32),
        pltpu.VMEM((CHUNK, AD_COLS), jnp.float32),
        pltpu.VMEM((CHUNK, SRC_COLS), jnp.float32),
        pltpu.VMEM((RU, SRC_COLS), jnp.float32),
        pltpu.VMEM_SHARED((NPAD, SRC_COLS), jnp.float32),
        pltpu.SemaphoreType.DMA,
        pltpu.SemaphoreType.DMA,
        pltpu.SemaphoreType.DMA,
        pltpu.SemaphoreType.DMA,
        pltpu.SemaphoreType.DMA,
    ],
)(_edge_body)


def _combine_kernel(num_ref, whin_ref, b16_ref, op_ref, oa_ref, os_ref):
    b16 = b16_ref[...]
    ft = []
    for r in range(6):
        x = num_ref[r, 0] + num_ref[r, 1]
        n = x[:, 0:F]
        sv = x[:, F:SRC_COLS]
        recip = 1.0 / (sv + 1e-9)
        ft.append(n * jnp.dot(recip, b16, preferred_element_type=jnp.float32))
    op_ref[...] = jax.nn.relu(ft[0] + ft[2])
    oa_ref[...] = jax.nn.relu(ft[1] + ft[3])
    os_ref[...] = jax.nn.relu(ft[4] + ft[5] + whin_ref[...])


def _tc_combine(nums, whin, b16):
    grid = (N // BM,)
    out_sds = jax.ShapeDtypeStruct((N, F), jnp.float32)
    return pl.pallas_call(
        _combine_kernel,
        grid=grid,
        in_specs=[
            pl.BlockSpec((6, NC, BM, SRC_COLS), lambda i: (0, 0, i, 0)),
            pl.BlockSpec((BM, F), lambda i: (i, 0)),
            pl.BlockSpec((AD_COLS, F), lambda i: (0, 0)),
        ],
        out_specs=[
            pl.BlockSpec((BM, F), lambda i: (i, 0)),
            pl.BlockSpec((BM, F), lambda i: (i, 0)),
            pl.BlockSpec((BM, F), lambda i: (i, 0)),
        ],
        out_shape=[out_sds, out_sds, out_sds],
    )(nums, whin, b16)


def _attn_cols(W, b, attn):
    a = attn[0]
    d = a.shape[1]
    v = jnp.einsum('khd,hd->kh', W.reshape(128, H, d), a)
    vb = jnp.einsum('hd,hd->h', b.reshape(H, d), a)
    return v, vb


def _type_block(W_rel, b_rel, attn_src):
    v, vb = _attn_cols(W_rel, b_rel, attn_src)
    z = jnp.zeros((128, 12), jnp.float32)
    zb = jnp.zeros((12,), jnp.float32)
    return (jnp.concatenate([W_rel, v, z], axis=1),
            jnp.concatenate([b_rel, vb, zb]))


def _ad_block(W_t, b_t, attn_dst):
    v, vb = _attn_cols(W_t, b_t, attn_dst)
    z = jnp.zeros((128, 12), jnp.float32)
    zb = jnp.zeros((12,), jnp.float32)
    return jnp.concatenate([v, z], axis=1), jnp.concatenate([vb, zb])


def kernel(feat_P, feat_A, feat_state, edge_p2p, edge_p2a, edge_a2p, edge_a2a, edge_p2s, edge_a2s, W_P, b_P, W_A, b_A, W_p2p, b_p2p, W_p2a, b_p2a, W_a2p, b_a2p, W_a2a, b_a2a, W_p2s, b_p2s, W_a2s, b_a2s, W_in, b_in, attn_src_p2p, attn_dst_p2p, attn_src_p2a, attn_dst_p2a, attn_src_a2p, attn_dst_a2p, attn_src_a2a, attn_dst_a2a, attn_src_p2s, attn_dst_p2s, attn_src_a2s, attn_dst_a2s):
    f32 = jnp.float32

    zpad = jnp.zeros((128, 48), f32)
    zpadb = jnp.zeros((48,), f32)
    zblk = jnp.zeros((128, 144), f32)
    zblkb = jnp.zeros((144,), f32)

    bp0, bbp0 = _type_block(W_p2p, b_p2p, attn_src_p2p)
    bp1, bbp1 = _type_block(W_p2a, b_p2a, attn_src_p2a)
    bp2, bbp2 = _type_block(W_p2s, b_p2s, attn_src_p2s)
    adP0, adbP0 = _ad_block(W_P, b_P, attn_dst_p2p)
    adP1, adbP1 = _ad_block(W_P, b_P, attn_dst_a2p)
    WeP = jnp.concatenate([bp0, bp1, bp2, adP0, adP1, zpad], axis=1)
    beP = jnp.concatenate([bbp0, bbp1, bbp2, adbP0, adbP1, zpadb])

    ba0, bba0 = _type_block(W_a2p, b_a2p, attn_src_a2p)
    ba1, bba1 = _type_block(W_a2a, b_a2a, attn_src_a2a)
    ba2, bba2 = _type_block(W_a2s, b_a2s, attn_src_a2s)
    adA0, adbA0 = _ad_block(W_A, b_A, attn_dst_p2a)
    adA1, adbA1 = _ad_block(W_A, b_A, attn_dst_a2a)
    WeA = jnp.concatenate([ba0, ba1, ba2, adA0, adA1, zpad], axis=1)
    beA = jnp.concatenate([bba0, bba1, bba2, adbA0, adbA1, zpadb])

    bs0 = jnp.concatenate([W_in, jnp.zeros((128, 16), f32)], axis=1)
    bbs0 = jnp.concatenate([b_in, jnp.zeros((16,), f32)])
    adS0, adbS0 = _ad_block(W_in, b_in, attn_dst_p2s)
    adS1, adbS1 = _ad_block(W_in, b_in, attn_dst_a2s)
    WeS = jnp.concatenate([bs0, zblk, zblk, adS0, adS1, zpad], axis=1)
    beS = jnp.concatenate([bbs0, zblkb, zblkb, adbS0, adbS1, zpadb])

    W_eff = jnp.stack([WeP, WeA, WeS])
    b_eff = jnp.stack([beP, beA, beS])[:, None, :]
    feats = jnp.stack([feat_P, feat_A, feat_state])

    big = _tc_matmul(feats, W_eff, b_eff)
    bigP, bigA, bigS = big[0], big[1], big[2]

    src_tab = jnp.stack([
        bigP[:, 0:144], bigP[:, 144:288],
        bigA[:, 0:144], bigA[:, 144:288],
        bigP[:, 288:432], bigA[:, 288:432],
    ])
    ad_tab = jnp.stack([
        bigP[:, 432:448], bigA[:, 432:448],
        bigP[:, 448:464], bigA[:, 448:464],
        bigS[:, 432:448], bigS[:, 448:464],
    ])
    ad_tab = jnp.concatenate(
        [ad_tab, jnp.zeros((6, NPAD - N, AD_COLS), f32)], axis=1)
    whin = bigS[:, 0:128]

    edges = jnp.stack([edge_p2p, edge_p2a, edge_a2p,
                       edge_a2a, edge_p2s, edge_a2s])
    per_sc = E // NC
    npad_e = NS * CPT * CHUNK - per_sc
    dummy_vals = jnp.array([0, N], jnp.int32)
    eh = edges.reshape(6, 2, NC, per_sc)
    pad1 = jnp.broadcast_to(dummy_vals[None, :, None, None],
                            (6, 2, NC, npad_e))
    eh = jnp.concatenate([eh, pad1], axis=3)
    eh = eh.reshape(6, 2, NC, NS, CPT, CHUNK)
    pad2 = jnp.broadcast_to(dummy_vals[None, :, None, None, None, None],
                            (6, 2, NC, NS, NPREF, CHUNK))
    eh = jnp.concatenate([eh, pad2], axis=4)
    edges = eh.transpose(0, 2, 3, 4, 1, 5).reshape(-1)

    nums = _edge_kernel(edges, src_tab, ad_tab)

    b16 = np.zeros((AD_COLS, F), np.float32)
    for h in range(H):
        b16[h, h * D:(h + 1) * D] = 1.0
    out_P, out_A, out_S = _tc_combine(nums, whin, jnp.asarray(b16))

    return (out_P.reshape(N, H, D),
            out_A.reshape(N, H, D),
            out_S.reshape(N, H, D))
